# baseline (speedup 1.0000x reference)
import functools

import jax
import jax.numpy as jnp
from jax import lax
from jax.experimental import pallas as pl
from jax.experimental.pallas import tpu as pltpu
from jax.experimental.pallas import tpu_sc as plsc

N = 10000
E = 320000
R = 4
IN_SIZE = 128
OUT_SIZE = 128

NC = 2
NS = 16
NW = NC * NS

CH = 128
NB = 2
TOTAL_E = R * E
CHUNKS_PER_W = 320
NG = CHUNKS_PER_W // NB
PW = CHUNKS_PER_W * CH
TP = PW * NW
TPC = TP // CH
NPAD = 10112
DUMMY_DST = 10000
ZROWS = NPAD // NS


def _h_body(x_ref, w_ref, h_ref):
    h_ref[...] = jnp.dot(x_ref[...], w_ref[0], preferred_element_type=jnp.float32)


def _combine_body(p_ref, b_ref, o_ref):
    bias_sum = jnp.sum(b_ref[...], axis=0, keepdims=True)
    o_ref[...] = p_ref[0, :N] + p_ref[1, :N] + bias_sum


def _sc_agg(h, idx3d):
    mesh = plsc.VectorSubcoreMesh(core_axis_name="c", subcore_axis_name="s")

    @functools.partial(
        pl.kernel,
        mesh=mesh,
        out_type=jax.ShapeDtypeStruct((NC, NPAD, OUT_SIZE), jnp.float32),
        scratch_types=[
            pltpu.VMEM((NB, 2, CH), jnp.int32),
            pltpu.VMEM((NB, 2, CH), jnp.int32),
            pltpu.VMEM((CH, OUT_SIZE), jnp.float32),
            pltpu.VMEM((CH, OUT_SIZE), jnp.float32),
            pltpu.VMEM_SHARED((NPAD, OUT_SIZE), jnp.float32),
            pltpu.SemaphoreType.DMA,
            pltpu.SemaphoreType.DMA,
            pltpu.SemaphoreType.DMA,
            pltpu.SemaphoreType.DMA,
            pltpu.SemaphoreType.DMA,
            pltpu.SemaphoreType.DMA,
        ],
    )
    def sc_fn(h_hbm, idx_hbm, part_hbm,
              idx0, idx1, rows0, rows1, acc,
              sem_i0, sem_i1, sg0, sg1, ss0, ss1):
        rows = [rows0, rows1]
        sg = [sg0, sg1]
        ss = [ss0, ss1]
        cid = lax.axis_index("c")
        sid = lax.axis_index("s")
        wid = cid * NS + sid
        cbase = wid * CHUNKS_PER_W

        @pl.loop(0, CH)
        def _(i):
            @pl.loop(0, OUT_SIZE, step=16)
            def _(j):
                rows0[i, pl.ds(j, 16)] = jnp.zeros((16,), jnp.float32)

        zbase = sid * ZROWS
        @pl.loop(0, ZROWS // CH)
        def _(k):
            pltpu.sync_copy(rows0, acc.at[pl.ds(zbase + k * CH, CH)])
        pltpu.sync_copy(rows0.at[pl.ds(0, ZROWS % CH)],
                        acc.at[pl.ds(zbase + (ZROWS // CH) * CH, ZROWS % CH)])
        plsc.subcore_barrier()

        def idx_fire(g, buf, sem):
            return pltpu.async_copy(idx_hbm.at[pl.ds((cbase + g * NB), NB)],
                                    buf, sem)

        idx_fire(0, idx0, sem_i0)

        @pl.loop(0, NG, step=2)
        def _(g0):
            def do_group(g, ib, prev_scat):
                pltpu.make_async_copy(
                    idx_hbm.at[pl.ds(cbase + g * NB, NB)], ib,
                    sem_i0 if ib is idx0 else sem_i1).wait()
                gathers = []
                for b in range(NB):
                    if prev_scat is not None:
                        prev_scat[b].wait()
                    gathers.append(
                        pltpu.async_copy(h_hbm.at[ib.at[b, 0]], rows[b], sg[b]))
                scats = []
                for b in range(NB):
                    gathers[b].wait()
                    scats.append(
                        pltpu.async_copy(rows[b], acc.at[ib.at[b, 1]], ss[b],
                                         add=True))
                return scats

            idx_fire(g0 + 1, idx1, sem_i1)
            s_a = do_group(g0, idx0, None)
            @pl.when(g0 + 2 < NG)
            def _():
                idx_fire(g0 + 2, idx0, sem_i0)
            s_b = do_group(g0 + 1, idx1, s_a)
            for b in range(NB):
                s_b[b].wait()

        plsc.subcore_barrier()
        pltpu.sync_copy(acc.at[pl.ds(sid * ZROWS, ZROWS)],
                        part_hbm.at[cid, pl.ds(sid * ZROWS, ZROWS)])

    return sc_fn(h, idx3d)


def kernel(inp, edge_index, weights, bias):
    h = pl.pallas_call(
        _h_body,
        grid=(R, N // 1000),
        in_specs=[
            pl.BlockSpec((1000, IN_SIZE), lambda r, i: (i, 0)),
            pl.BlockSpec((1, IN_SIZE, OUT_SIZE), lambda r, i: (r, 0, 0)),
        ],
        out_specs=pl.BlockSpec((1000, OUT_SIZE), lambda r, i: (r * (N // 1000) + i, 0)),
        out_shape=jax.ShapeDtypeStruct((R * N, OUT_SIZE), jnp.float32),
    )(inp, weights)

    rel_off = (jnp.arange(R, dtype=jnp.int32) * N)[:, None]
    src_flat = (edge_index[:, 1, :] + rel_off).reshape(-1)
    dst_flat = edge_index[:, 0, :].reshape(-1)
    pad = TP - TOTAL_E
    src_flat = jnp.concatenate([src_flat, jnp.zeros((pad,), jnp.int32)])
    dst_flat = jnp.concatenate([dst_flat, jnp.full((pad,), DUMMY_DST, jnp.int32)])
    idx3d = jnp.stack([src_flat.reshape(TPC, CH), dst_flat.reshape(TPC, CH)],
                      axis=1)

    part = _sc_agg(h, idx3d)

    out = pl.pallas_call(
        _combine_body,
        out_shape=jax.ShapeDtypeStruct((N, OUT_SIZE), jnp.float32),
    )(part, bias)
    return out
